# baseline (speedup 1.0000x reference)
import functools

import jax
import jax.numpy as jnp
from jax import lax
from jax.experimental import pallas as pl
from jax.experimental.pallas import tpu as pltpu
from jax.experimental.pallas import tpu_sc as plsc

_N = 10000
_NFEAT = 128
_NHID = 32
_B = 16384
_BTOT = 3 * _B

_BM = 400
_NBLK = _N // _BM


def _gcn_tc_body(x_ref, w1_ref, b1_ref, w2_ref, b2_ref, adj_ref, h_ref,
                 s1_scr, s2_scr):
    p = pl.program_id(0)
    i = pl.program_id(1)

    @pl.when((p == 0) & (i == 0))
    def _():
        s1_scr[...] = jnp.dot(x_ref[...], w1_ref[...],
                              preferred_element_type=jnp.float32)

    @pl.when(p == 0)
    def _():
        h1 = jnp.dot(adj_ref[...], s1_scr[...],
                     preferred_element_type=jnp.float32) + b1_ref[...]
        h1 = jnp.maximum(h1, 0.0)
        s2_scr[pl.ds(i * _BM, _BM), :] = jnp.dot(
            h1, w2_ref[...], preferred_element_type=jnp.float32)

    @pl.when(p == 1)
    def _():
        h_ref[pl.ds(i * _BM, _BM), :] = jnp.dot(
            adj_ref[...], s2_scr[...],
            preferred_element_type=jnp.float32) + b2_ref[...]


def _gcn_tc(x, adj, W1, b1, W2, b2):
    return pl.pallas_call(
        _gcn_tc_body,
        grid=(2, _NBLK),
        in_specs=[
            pl.BlockSpec((_N, _NFEAT), lambda p, i: (0, 0)),
            pl.BlockSpec((_NFEAT, _NHID), lambda p, i: (0, 0)),
            pl.BlockSpec((1, _NHID), lambda p, i: (0, 0)),
            pl.BlockSpec((_NHID, _NHID), lambda p, i: (0, 0)),
            pl.BlockSpec((1, _NHID), lambda p, i: (0, 0)),
            pl.BlockSpec((_BM, _N), lambda p, i: (i, 0)),
        ],
        out_specs=pl.BlockSpec((_N, _NHID), lambda p, i: (0, 0)),
        out_shape=jax.ShapeDtypeStruct((_N, _NHID), jnp.float32),
        scratch_shapes=[
            pltpu.VMEM((_N, _NHID), jnp.float32),
            pltpu.VMEM((_N, _NHID), jnp.float32),
        ],
    )(x, W1, b1, W2, b2, adj)


_SC_INFO = plsc.get_sparse_core_info()
_NW = _SC_INFO.num_cores * _SC_INFO.num_subcores
_B_PER_W = _BTOT // _NW
_CHUNK = 128
_NCHUNK = _B_PER_W // _CHUNK


def _sc_gather_body(table_hbm, idx_hbm, out_hbm, idx_v, rows_v, sem):
    wid = lax.axis_index("s") * _SC_INFO.num_cores + lax.axis_index("c")
    base = wid * _B_PER_W
    pltpu.sync_copy(idx_hbm.at[pl.ds(base, _B_PER_W)], idx_v)
    copies = []
    for j in range(_NCHUNK):
        copies.append(pltpu.async_copy(
            table_hbm.at[idx_v.at[pl.ds(j * _CHUNK, _CHUNK)]],
            rows_v.at[j % 2], sem))
        if j >= 1:
            copies[j - 1].wait()
            pltpu.sync_copy(rows_v.at[(j - 1) % 2],
                            out_hbm.at[pl.ds(base + (j - 1) * _CHUNK, _CHUNK)])
    copies[-1].wait()
    pltpu.sync_copy(rows_v.at[(_NCHUNK - 1) % 2],
                    out_hbm.at[pl.ds(base + (_NCHUNK - 1) * _CHUNK, _CHUNK)])


@functools.partial(
    pl.kernel,
    mesh=plsc.VectorSubcoreMesh(core_axis_name="c", subcore_axis_name="s"),
    out_type=jax.ShapeDtypeStruct((_BTOT, _NHID), jnp.float32),
    scratch_types=[
        pltpu.VMEM((_B_PER_W,), jnp.int32),
        pltpu.VMEM((2, _CHUNK, _NHID), jnp.float32),
        pltpu.SemaphoreType.DMA,
    ],
    compiler_params=pltpu.CompilerParams(use_tc_tiling_on_sc=False),
)
def _sc_gather(table_hbm, idx_hbm, out_hbm, idx_v, rows_v, sem):
    _sc_gather_body(table_hbm, idx_hbm, out_hbm, idx_v, rows_v, sem)


_LB = 4096
_LBLK = _B // _LB


def _loss_body(he_ref, pe_ref, ne_ref, out_ref, acc):
    i = pl.program_id(0)
    he = he_ref[...]
    pe = pe_ref[...]
    ne = ne_ref[...]
    ps = jnp.sum(he * pe, axis=1)
    ns = jnp.sum(he * ne, axis=1)
    z = ps - ns
    sp = jnp.maximum(-z, 0.0) + jnp.log(1.0 + jnp.exp(-jnp.abs(z)))
    part = jnp.sum(sp) + (1e-5 * 0.5) * (
        jnp.sum(he * he) + jnp.sum(pe * pe) + jnp.sum(ne * ne))

    @pl.when(i == 0)
    def _():
        acc[0] = 0.0

    acc[0] += part

    @pl.when(i == _LBLK - 1)
    def _():
        out_ref[0, 0] = acc[0] * (1.0 / _B)


def _loss_tc(emb):
    out = pl.pallas_call(
        _loss_body,
        grid=(_LBLK,),
        in_specs=[
            pl.BlockSpec((_LB, _NHID), lambda i: (i, 0)),
            pl.BlockSpec((_LB, _NHID), lambda i: (i + _LBLK, 0)),
            pl.BlockSpec((_LB, _NHID), lambda i: (i + 2 * _LBLK, 0)),
        ],
        out_specs=pl.BlockSpec(memory_space=pltpu.SMEM),
        out_shape=jax.ShapeDtypeStruct((1, 1), jnp.float32),
        scratch_shapes=[pltpu.SMEM((1,), jnp.float32)],
    )(emb, emb, emb)
    return out[0, 0]


def kernel(x, adj, head, pos_tail, neg_tail, W1, b1, W2, b2):
    idx_all = jnp.concatenate([
        head.astype(jnp.int32),
        pos_tail.astype(jnp.int32),
        neg_tail.astype(jnp.int32),
    ])
    h = _gcn_tc(x, adj, W1, b1.reshape(1, _NHID), W2, b2.reshape(1, _NHID))
    emb = _sc_gather(h, idx_all)
    return _loss_tc(emb)


# speedup vs baseline: 1.0650x; 1.0650x over previous
import functools

import jax
import jax.numpy as jnp
from jax import lax
from jax.experimental import pallas as pl
from jax.experimental.pallas import tpu as pltpu
from jax.experimental.pallas import tpu_sc as plsc

_N = 10000
_NFEAT = 128
_NHID = 32
_HPAD = 128
_B = 16384
_BTOT = 3 * _B
_PACK = _HPAD // _NHID

_BM = 400
_NBLK = _N // _BM


def _gcn_tc_body(x_ref, w1_ref, b1_ref, w2_ref, b2_ref, adj_ref, h_ref,
                 s1_scr, s2_scr):
    p = pl.program_id(0)
    i = pl.program_id(1)

    @pl.when((p == 0) & (i == 0))
    def _():
        s1_scr[...] = jnp.dot(x_ref[...], w1_ref[...],
                              preferred_element_type=jnp.float32)

    @pl.when(p == 0)
    def _():
        h1 = jnp.dot(adj_ref[...], s1_scr[...],
                     preferred_element_type=jnp.float32) + b1_ref[...]
        h1 = jnp.maximum(h1, 0.0)
        s2_scr[pl.ds(i * _BM, _BM), :] = jnp.dot(
            h1, w2_ref[...], preferred_element_type=jnp.float32)

    @pl.when(p == 1)
    def _():
        h_ref[pl.ds(i * _BM, _BM), :] = jnp.dot(
            adj_ref[...], s2_scr[...],
            preferred_element_type=jnp.float32) + b2_ref[...]


def _gcn_tc(x, adj, W1, b1, W2p, b2p):
    return pl.pallas_call(
        _gcn_tc_body,
        grid=(2, _NBLK),
        in_specs=[
            pl.BlockSpec((_N, _NFEAT), lambda p, i: (0, 0)),
            pl.BlockSpec((_NFEAT, _NHID), lambda p, i: (0, 0)),
            pl.BlockSpec((1, _NHID), lambda p, i: (0, 0)),
            pl.BlockSpec((_NHID, _HPAD), lambda p, i: (0, 0)),
            pl.BlockSpec((1, _HPAD), lambda p, i: (0, 0)),
            pl.BlockSpec((_BM, _N), lambda p, i: (i, 0)),
        ],
        out_specs=pl.BlockSpec((_N, _HPAD), lambda p, i: (0, 0)),
        out_shape=jax.ShapeDtypeStruct((_N, _HPAD), jnp.float32),
        scratch_shapes=[
            pltpu.VMEM((_N, _NHID), jnp.float32),
            pltpu.VMEM((_N, _HPAD), jnp.float32),
        ],
    )(x, W1, b1, W2p, b2p, adj)


_SC_INFO = plsc.get_sparse_core_info()
_NW = _SC_INFO.num_cores * _SC_INFO.num_subcores
_B_PER_W = _BTOT // _NW
_CHUNK = 128
_NCHUNK = _B_PER_W // _CHUNK


def _sc_gather_body(table_hbm, idx_hbm, out_hbm, idx_v, rows_v, sem):
    wid = lax.axis_index("s") * _SC_INFO.num_cores + lax.axis_index("c")
    base = wid * _B_PER_W
    pltpu.sync_copy(idx_hbm.at[pl.ds(base, _B_PER_W)], idx_v)
    copies = []
    for j in range(_NCHUNK):
        copies.append(pltpu.async_copy(
            table_hbm.at[idx_v.at[pl.ds(j * _CHUNK, _CHUNK)]],
            rows_v.at[j % 2], sem))
        if j >= 1:
            copies[j - 1].wait()
            pltpu.sync_copy(rows_v.at[(j - 1) % 2],
                            out_hbm.at[pl.ds(base + (j - 1) * _CHUNK, _CHUNK)])
    copies[-1].wait()
    pltpu.sync_copy(rows_v.at[(_NCHUNK - 1) % 2],
                    out_hbm.at[pl.ds(base + (_NCHUNK - 1) * _CHUNK, _CHUNK)])


@functools.partial(
    pl.kernel,
    mesh=plsc.VectorSubcoreMesh(core_axis_name="c", subcore_axis_name="s"),
    out_type=jax.ShapeDtypeStruct((_BTOT, _NHID), jnp.float32),
    scratch_types=[
        pltpu.VMEM((_B_PER_W,), jnp.int32),
        pltpu.VMEM((2, _CHUNK, _NHID), jnp.float32),
        pltpu.SemaphoreType.DMA,
    ],
    compiler_params=pltpu.CompilerParams(use_tc_tiling_on_sc=False),
)
def _sc_gather(table_hbm, idx_hbm, out_hbm, idx_v, rows_v, sem):
    _sc_gather_body(table_hbm, idx_hbm, out_hbm, idx_v, rows_v, sem)


_LB = 1024
_LBLK = _B // (_LB * _PACK)


def _softplus_neg(z):
    return jnp.maximum(-z, 0.0) + jnp.log(1.0 + jnp.exp(-jnp.abs(z)))


def _loss_body(he_ref, pe_ref, ne_ref, out_ref, acc):
    i = pl.program_id(0)
    he = he_ref[...]
    pe = pe_ref[...]
    ne = ne_ref[...]
    qp = he * pe
    qn = he * ne
    bce = 0.0
    for g in range(_PACK):
        sl = slice(g * _NHID, (g + 1) * _NHID)
        zg = jnp.sum(qp[:, sl], axis=1) - jnp.sum(qn[:, sl], axis=1)
        bce += jnp.sum(_softplus_neg(zg))
    part = bce + (1e-5 * 0.5) * (
        jnp.sum(he * he) + jnp.sum(pe * pe) + jnp.sum(ne * ne))

    @pl.when(i == 0)
    def _():
        acc[0] = 0.0

    acc[0] += part

    @pl.when(i == _LBLK - 1)
    def _():
        out_ref[0, 0] = acc[0] * (1.0 / _B)


def _loss_tc(emb):
    out = pl.pallas_call(
        _loss_body,
        grid=(_LBLK,),
        in_specs=[
            pl.BlockSpec((_LB, _HPAD), lambda i: (i, 0)),
            pl.BlockSpec((_LB, _HPAD), lambda i: (i + _LBLK, 0)),
            pl.BlockSpec((_LB, _HPAD), lambda i: (i + 2 * _LBLK, 0)),
        ],
        out_specs=pl.BlockSpec(memory_space=pltpu.SMEM),
        out_shape=jax.ShapeDtypeStruct((1, 1), jnp.float32),
        scratch_shapes=[pltpu.SMEM((1,), jnp.float32)],
    )(emb, emb, emb)
    return out[0, 0]


def kernel(x, adj, head, pos_tail, neg_tail, W1, b1, W2, b2):
    idx_all = _PACK * jnp.concatenate([
        head.astype(jnp.int32),
        pos_tail.astype(jnp.int32),
        neg_tail.astype(jnp.int32),
    ])
    W2p = jnp.pad(W2, ((0, 0), (0, _HPAD - _NHID)))
    b2p = jnp.pad(b2, (0, _HPAD - _NHID)).reshape(1, _HPAD)
    h = _gcn_tc(x, adj, W1, b1.reshape(1, _NHID), W2p, b2p)
    table = h.reshape(_N * _PACK, _NHID)
    emb = _sc_gather(table, idx_all)
    return _loss_tc(emb.reshape(_BTOT // _PACK, _HPAD))
